# Initial kernel scaffold; baseline (speedup 1.0000x reference)
import functools

import jax
import jax.numpy as jnp
from jax import lax
from jax.experimental import pallas as pl
from jax.experimental.pallas import tpu as pltpu
from jax.experimental.pallas import tpu_sc as plsc

N = 10000
E = 320000
D = 128
G = 64

NC = 2
NS = 16
NW = NC * NS

CHUNK = 128
CPT = 79
EPT = CPT * CHUNK
E_PAD = NW * EPT
N_PAD = 10240
ROWS_PT = N_PAD // NS
ZCHUNKS = ROWS_PT // CHUNK


def _sc_agg_body(h_hbm, src_hbm, dst_hbm, zeros_hbm, out_hbm,
                 src_v, dst_v, rows_v, zero_v, agg_sh):
    c = lax.axis_index("c")
    s = lax.axis_index("s")
    wid = c * NS + s

    pltpu.sync_copy(zeros_hbm, zero_v)
    for k in range(ZCHUNKS):
        pltpu.sync_copy(zero_v,
                        agg_sh.at[pl.ds(s * ROWS_PT + k * CHUNK, CHUNK)])
    pltpu.sync_copy(src_hbm.at[wid], src_v)
    pltpu.sync_copy(dst_hbm.at[wid], dst_v)
    plsc.subcore_barrier()

    def body(j, carry):
        pltpu.sync_copy(h_hbm.at[src_v.at[j]], rows_v)
        pltpu.sync_copy(rows_v, agg_sh.at[dst_v.at[j]], add=True)
        return carry

    lax.fori_loop(0, CPT, body, 0)
    plsc.subcore_barrier()

    pltpu.sync_copy(agg_sh.at[pl.ds(s * ROWS_PT, ROWS_PT)],
                    out_hbm.at[c, pl.ds(s * ROWS_PT, ROWS_PT)])


_sc_agg = pl.kernel(
    _sc_agg_body,
    out_type=jax.ShapeDtypeStruct((NC, N_PAD, D), jnp.float32),
    mesh=plsc.VectorSubcoreMesh(core_axis_name="c", subcore_axis_name="s"),
    scratch_types=[
        pltpu.VMEM((CPT, CHUNK), jnp.int32),
        pltpu.VMEM((CPT, CHUNK), jnp.int32),
        pltpu.VMEM((CHUNK, D), jnp.float32),
        pltpu.VMEM((CHUNK, D), jnp.float32),
        pltpu.VMEM_SHARED((N_PAD, D), jnp.float32),
    ],
)


def _mlp_bn(h_ref, agg_ref, W1_ref, b1_ref, g_ref, be_ref, W2_ref, b2_ref):
    hsum = h_ref[...] + agg_ref[0, :N, :] + agg_ref[1, :N, :]
    z = jnp.dot(hsum, W1_ref[...], preferred_element_type=jnp.float32)
    z = jnp.maximum(z + b1_ref[...], 0.0)
    mu = jnp.mean(z, axis=0, keepdims=True)
    var = jnp.mean((z - mu) ** 2, axis=0, keepdims=True)
    zn = (z - mu) / jnp.sqrt(var + 1e-5) * g_ref[...] + be_ref[...]
    h2 = jnp.dot(zn, W2_ref[...], preferred_element_type=jnp.float32)
    return jnp.maximum(h2 + b2_ref[...], 0.0)


def _tc_layer_body(h_ref, agg_ref, W1_ref, b1_ref, g_ref, be_ref,
                   W2_ref, b2_ref, out_ref):
    out_ref[...] = _mlp_bn(h_ref, agg_ref, W1_ref, b1_ref, g_ref, be_ref,
                           W2_ref, b2_ref)


def _tc_layer_pool_body(h_ref, agg_ref, batch_ref, W1_ref, b1_ref, g_ref,
                        be_ref, W2_ref, b2_ref, out_ref):
    h3 = _mlp_bn(h_ref, agg_ref, W1_ref, b1_ref, g_ref, be_ref,
                 W2_ref, b2_ref)
    gids = lax.broadcasted_iota(jnp.int32, (G, N), 0)
    onehot = (gids == batch_ref[...]).astype(jnp.float32)
    out_ref[...] = jnp.dot(onehot, h3, preferred_element_type=jnp.float32)


_tc_layer = pl.pallas_call(
    _tc_layer_body,
    out_shape=jax.ShapeDtypeStruct((N, D), jnp.float32),
)

_tc_layer_pool = pl.pallas_call(
    _tc_layer_pool_body,
    out_shape=jax.ShapeDtypeStruct((G, D), jnp.float32),
)


def kernel(x, edge_index, batch, W1_0, b1_0, g_0, be_0, W2_0, b2_0,
           W1_1, b1_1, g_1, be_1, W2_1, b2_1,
           W1_2, b1_2, g_2, be_2, W2_2, b2_2):
    src = edge_index[0]
    dst = edge_index[1]
    pad = E_PAD - E
    src3 = jnp.concatenate([src, jnp.zeros((pad,), jnp.int32)]).reshape(
        NW, CPT, CHUNK)
    dst3 = jnp.concatenate([dst, jnp.full((pad,), N, jnp.int32)]).reshape(
        NW, CPT, CHUNK)
    zeros = jnp.zeros((CHUNK, D), jnp.float32)
    batch2 = batch.reshape(1, N)

    params = [
        (W1_0, b1_0, g_0, be_0, W2_0, b2_0),
        (W1_1, b1_1, g_1, be_1, W2_1, b2_1),
        (W1_2, b1_2, g_2, be_2, W2_2, b2_2),
    ]
    h = x
    for i, (W1, b1, g, be, W2, b2) in enumerate(params):
        agg = _sc_agg(h, src3, dst3, zeros)
        args = (h, agg, W1, b1.reshape(1, D), g.reshape(1, D),
                be.reshape(1, D), W2, b2.reshape(1, D))
        if i < 2:
            h = _tc_layer(*args)
        else:
            h = _tc_layer_pool(h, agg, batch2, *args[2:])
    return h


# speedup vs baseline: 4.3324x; 4.3324x over previous
import functools

import jax
import jax.numpy as jnp
from jax import lax
from jax.experimental import pallas as pl
from jax.experimental.pallas import tpu as pltpu
from jax.experimental.pallas import tpu_sc as plsc

N = 10000
E = 320000
D = 128
G = 64

NC = 2
NS = 16
NW = NC * NS

CHUNK = 128
CPT = 79
EPT = CPT * CHUNK
E_PAD = NW * EPT
N_PAD = 10240
ROWS_PT = N_PAD // NS
ZCHUNKS = ROWS_PT // CHUNK


def _sc_agg_body(h_hbm, src_hbm, dst_hbm, zeros_hbm, out_hbm,
                 src_v, dst_v, rows_v, agg_sh):
    c = lax.axis_index("c")
    s = lax.axis_index("s")
    wid = c * NS + s

    pltpu.sync_copy(zeros_hbm, rows_v)
    for k in range(ZCHUNKS):
        pltpu.sync_copy(rows_v,
                        agg_sh.at[pl.ds(s * ROWS_PT + k * CHUNK, CHUNK)])
    pltpu.sync_copy(src_hbm.at[wid], src_v)
    pltpu.sync_copy(dst_hbm.at[wid], dst_v)
    plsc.subcore_barrier()

    def body(j, carry):
        pltpu.sync_copy(h_hbm.at[src_v.at[j]], rows_v)
        pltpu.sync_copy(rows_v, agg_sh.at[dst_v.at[j]], add=True)
        return carry

    lax.fori_loop(0, CPT, body, 0)
    plsc.subcore_barrier()

    pltpu.sync_copy(agg_sh.at[pl.ds(s * ROWS_PT, ROWS_PT)],
                    out_hbm.at[c, pl.ds(s * ROWS_PT, ROWS_PT)])


_sc_agg = pl.kernel(
    _sc_agg_body,
    out_type=jax.ShapeDtypeStruct((NC, N_PAD, D), jnp.float32),
    mesh=plsc.VectorSubcoreMesh(core_axis_name="c", subcore_axis_name="s"),
    scratch_types=[
        pltpu.VMEM((CPT, CHUNK), jnp.int32),
        pltpu.VMEM((CPT, CHUNK), jnp.int32),
        pltpu.VMEM((CHUNK, D), jnp.float32),
        pltpu.VMEM_SHARED((N_PAD, D), jnp.float32),
    ],
)


def _mlp_bn(h_ref, agg_ref, W1_ref, b1_ref, g_ref, be_ref, W2_ref, b2_ref):
    hsum = h_ref[...] + agg_ref[0, :N, :] + agg_ref[1, :N, :]
    z = jnp.dot(hsum, W1_ref[...], preferred_element_type=jnp.float32)
    z = jnp.maximum(z + b1_ref[...], 0.0)
    mu = jnp.mean(z, axis=0, keepdims=True)
    var = jnp.mean((z - mu) ** 2, axis=0, keepdims=True)
    zn = (z - mu) / jnp.sqrt(var + 1e-5) * g_ref[...] + be_ref[...]
    h2 = jnp.dot(zn, W2_ref[...], preferred_element_type=jnp.float32)
    return jnp.maximum(h2 + b2_ref[...], 0.0)


def _tc_layer_body(h_ref, agg_ref, W1_ref, b1_ref, g_ref, be_ref,
                   W2_ref, b2_ref, out_ref):
    out_ref[...] = _mlp_bn(h_ref, agg_ref, W1_ref, b1_ref, g_ref, be_ref,
                           W2_ref, b2_ref)


def _tc_layer_pool_body(h_ref, agg_ref, batch_ref, W1_ref, b1_ref, g_ref,
                        be_ref, W2_ref, b2_ref, out_ref):
    h3 = _mlp_bn(h_ref, agg_ref, W1_ref, b1_ref, g_ref, be_ref,
                 W2_ref, b2_ref)
    gids = lax.broadcasted_iota(jnp.int32, (G, N), 0)
    onehot = (gids == batch_ref[...]).astype(jnp.float32)
    out_ref[...] = jnp.dot(onehot, h3, preferred_element_type=jnp.float32)


_tc_layer = pl.pallas_call(
    _tc_layer_body,
    out_shape=jax.ShapeDtypeStruct((N, D), jnp.float32),
)

_tc_layer_pool = pl.pallas_call(
    _tc_layer_pool_body,
    out_shape=jax.ShapeDtypeStruct((G, D), jnp.float32),
)


def kernel(x, edge_index, batch, W1_0, b1_0, g_0, be_0, W2_0, b2_0,
           W1_1, b1_1, g_1, be_1, W2_1, b2_1,
           W1_2, b1_2, g_2, be_2, W2_2, b2_2):
    src = edge_index[0]
    dst = edge_index[1]
    pad = E_PAD - E
    src3 = jnp.concatenate([src, jnp.zeros((pad,), jnp.int32)]).reshape(
        NW, CPT, CHUNK)
    dst3 = jnp.concatenate([dst, jnp.full((pad,), N, jnp.int32)]).reshape(
        NW, CPT, CHUNK)
    zeros = jnp.zeros((CHUNK, D), jnp.float32)
    batch2 = batch.reshape(1, N)

    params = [
        (W1_0, b1_0, g_0, be_0, W2_0, b2_0),
        (W1_1, b1_1, g_1, be_1, W2_1, b2_1),
        (W1_2, b1_2, g_2, be_2, W2_2, b2_2),
    ]
    h = x
    for i, (W1, b1, g, be, W2, b2) in enumerate(params):
        agg = _sc_agg(h, src3, dst3, zeros)
        args = (h, agg, W1, b1.reshape(1, D), g.reshape(1, D),
                be.reshape(1, D), W2, b2.reshape(1, D))
        if i < 2:
            h = _tc_layer(*args)
        else:
            h = _tc_layer_pool(h, agg, batch2, *args[2:])
    return h
